# Initial kernel scaffold; baseline (speedup 1.0000x reference)
import jax
import jax.numpy as jnp
from jax.experimental import pallas as pl


def kernel(logits):
    raise NotImplementedError("write your pallas kernel here")


# speedup vs baseline: 17.0185x; 17.0185x over previous
import numpy as np
import jax
import jax.numpy as jnp
from jax.experimental import pallas as pl

_B = 64
_V = 100000
_K = int((1 - 0.9) * _V)
_ROWS = 8
_GRID = _B // _ROWS

_KEY0 = 0
_KEY1 = 1


def _threefry_gumbel(flat_idx):
    ks0 = jnp.uint32(_KEY0)
    ks1 = jnp.uint32(_KEY1)
    ks2 = jnp.uint32(_KEY0 ^ _KEY1 ^ 0x1BD11BDA)
    ks = (ks0, ks1, ks2)
    rots = ((13, 15, 26, 6), (17, 29, 16, 24))

    x0 = jnp.zeros_like(flat_idx) + ks0
    x1 = flat_idx + ks1
    for i in range(5):
        for r in rots[i % 2]:
            x0 = x0 + x1
            x1 = (x1 << jnp.uint32(r)) | (x1 >> jnp.uint32(32 - r))
            x1 = x1 ^ x0
        x0 = x0 + ks[(i + 1) % 3]
        x1 = x1 + ks[(i + 2) % 3] + jnp.uint32(i + 1)
    bits = x0 ^ x1

    f = jax.lax.bitcast_convert_type(
        (bits >> jnp.uint32(9)) | jnp.uint32(0x3F800000), jnp.float32
    ) - jnp.float32(1.0)
    tiny = jnp.float32(np.finfo(np.float32).tiny)
    u = jnp.maximum(tiny, f + tiny)
    return -jnp.log(-jnp.log(u))


def _body(x_ref, probs_ref, samp_ref):
    pid = pl.program_id(0)
    x = x_ref[...]

    u = jax.lax.bitcast_convert_type(x, jnp.uint32)
    neg = u >= jnp.uint32(0x80000000)
    ukey = jnp.where(neg, ~u, u | jnp.uint32(0x80000000))

    prefix = jnp.zeros((_ROWS, 1), jnp.uint32)
    for b in range(31, -1, -1):
        cand = prefix | jnp.uint32(2**b)
        cnt = jnp.sum((ukey >= cand).astype(jnp.int32), axis=1, keepdims=True)
        prefix = jnp.where(cnt >= _K, cand, prefix)
    mask = ukey >= prefix

    rowmax = jnp.max(x, axis=1, keepdims=True)
    e = jnp.where(mask, jnp.exp(x - rowmax), jnp.float32(0.0))
    z = jnp.sum(e, axis=1, keepdims=True)
    probs_ref[...] = e / z

    col = jax.lax.broadcasted_iota(jnp.uint32, (_ROWS, _V), 1)
    row = jax.lax.broadcasted_iota(jnp.uint32, (_ROWS, _V), 0)
    flat = (jnp.uint32(pid * _ROWS) + row) * jnp.uint32(_V) + col
    g = _threefry_gumbel(flat)
    val = jnp.where(mask, x + g, jnp.float32(-np.inf))
    vmax = jnp.max(val, axis=1, keepdims=True)
    coli = jax.lax.broadcasted_iota(jnp.int32, (_ROWS, _V), 1)
    idx = jnp.min(jnp.where(val == vmax, coli, jnp.int32(_V)), axis=1)
    samp_ref[...] = idx.reshape(_ROWS, 1)


def kernel(logits):
    probs, samp = pl.pallas_call(
        _body,
        grid=(_GRID,),
        in_specs=[pl.BlockSpec((_ROWS, _V), lambda i: (i, 0))],
        out_specs=[
            pl.BlockSpec((_ROWS, _V), lambda i: (i, 0)),
            pl.BlockSpec((_ROWS, 1), lambda i: (i, 0)),
        ],
        out_shape=[
            jax.ShapeDtypeStruct((_B, _V), jnp.float32),
            jax.ShapeDtypeStruct((_B, 1), jnp.int32),
        ],
    )(logits)
    return probs, samp


# speedup vs baseline: 17.0266x; 1.0005x over previous
import numpy as np
import jax
import jax.numpy as jnp
from jax.experimental import pallas as pl
from jax.experimental.pallas import tpu as pltpu

_B = 64
_V = 100000
_K = int((1 - 0.9) * _V)
_ROWS = 8
_GRID = _B // _ROWS

_KEY0 = 0
_KEY1 = 1


def _threefry_gumbel(flat_idx):
    ks0 = jnp.uint32(_KEY0)
    ks1 = jnp.uint32(_KEY1)
    ks2 = jnp.uint32(_KEY0 ^ _KEY1 ^ 0x1BD11BDA)
    ks = (ks0, ks1, ks2)
    rots = ((13, 15, 26, 6), (17, 29, 16, 24))

    x0 = jnp.zeros_like(flat_idx) + ks0
    x1 = flat_idx + ks1
    for i in range(5):
        for r in rots[i % 2]:
            x0 = x0 + x1
            x1 = (x1 << jnp.uint32(r)) | (x1 >> jnp.uint32(32 - r))
            x1 = x1 ^ x0
        x0 = x0 + ks[(i + 1) % 3]
        x1 = x1 + ks[(i + 2) % 3] + jnp.uint32(i + 1)
    bits = x0 ^ x1

    f = jax.lax.bitcast_convert_type(
        (bits >> jnp.uint32(9)) | jnp.uint32(0x3F800000), jnp.float32
    ) - jnp.float32(1.0)
    tiny = jnp.float32(np.finfo(np.float32).tiny)
    u = jnp.maximum(tiny, f + tiny)
    return -jnp.log(-jnp.log(u))


def _body(x_ref, probs_ref, samp_ref):
    pid = pl.program_id(0)
    x = x_ref[...]

    u = jax.lax.bitcast_convert_type(x, jnp.uint32)
    neg = u >= jnp.uint32(0x80000000)
    ukey = jnp.where(neg, ~u, u | jnp.uint32(0x80000000))

    prefix = jnp.zeros((_ROWS, 1), jnp.uint32)
    for b in range(31, -1, -1):
        cand = prefix | jnp.uint32(2**b)
        cnt = jnp.sum((ukey >= cand).astype(jnp.int32), axis=1, keepdims=True)
        prefix = jnp.where(cnt >= _K, cand, prefix)
    mask = ukey >= prefix

    rowmax = jnp.max(x, axis=1, keepdims=True)
    e = jnp.where(mask, jnp.exp(x - rowmax), jnp.float32(0.0))
    z = jnp.sum(e, axis=1, keepdims=True)
    probs_ref[...] = e / z

    col = jax.lax.broadcasted_iota(jnp.uint32, (_ROWS, _V), 1)
    row = jax.lax.broadcasted_iota(jnp.uint32, (_ROWS, _V), 0)
    flat = (jnp.uint32(pid * _ROWS) + row) * jnp.uint32(_V) + col
    g = _threefry_gumbel(flat)
    val = jnp.where(mask, x + g, jnp.float32(-np.inf))
    vmax = jnp.max(val, axis=1, keepdims=True)
    coli = jax.lax.broadcasted_iota(jnp.int32, (_ROWS, _V), 1)
    idx = jnp.min(jnp.where(val == vmax, coli, jnp.int32(_V)), axis=1)
    samp_ref[...] = idx.reshape(_ROWS, 1)


def kernel(logits):
    probs, samp = pl.pallas_call(
        _body,
        grid=(_GRID,),
        in_specs=[pl.BlockSpec((_ROWS, _V), lambda i: (i, 0))],
        out_specs=[
            pl.BlockSpec((_ROWS, _V), lambda i: (i, 0)),
            pl.BlockSpec((_ROWS, 1), lambda i: (i, 0)),
        ],
        out_shape=[
            jax.ShapeDtypeStruct((_B, _V), jnp.float32),
            jax.ShapeDtypeStruct((_B, 1), jnp.int32),
        ],
        compiler_params=pltpu.CompilerParams(
            dimension_semantics=("parallel",),
        ),
    )(logits)
    return probs, samp
